# baseline (speedup 1.0000x reference)
import functools

import jax
import jax.numpy as jnp
from jax import lax
from jax.experimental import pallas as pl
from jax.experimental.pallas import tpu as pltpu
from jax.experimental.pallas import tpu_sc as plsc

_N = 10000
_E = 320000
_H = 128

_NC = 2
_NS = 16
_NW = _NC * _NS
_C = 128
_NCHUNK = 80
_EPW = _NCHUNK * _C
_EPAD = _NW * _EPW
_NPAD = 10240
_ZPS = (_NPAD // _C) // _NS

_BLK = 1000


def _mlp_body(x_ref, w_ref, b_ref, y_ref):
    y = jnp.dot(x_ref[...], w_ref[...], preferred_element_type=jnp.float32)
    y_ref[...] = jnp.maximum(y + b_ref[...], 0.0)


def _node_mlp(x, w1t, b1):
    return pl.pallas_call(
        _mlp_body,
        grid=(_N // _BLK,),
        in_specs=[
            pl.BlockSpec((_BLK, _H), lambda i: (i, 0)),
            pl.BlockSpec((_H, _H), lambda i: (0, 0)),
            pl.BlockSpec((1, _H), lambda i: (0, 0)),
        ],
        out_specs=pl.BlockSpec((_BLK, _H), lambda i: (i, 0)),
        out_shape=jax.ShapeDtypeStruct((_N, _H), jnp.float32),
    )(x, w1t, b1)


def _seg_sum_body(y_hbm, row_hbm, col_hbm, out_hbm,
                  ridx0, ridx1, cidx0, cidx1, rows0, rows1, acc,
                  sg0, sg1, sr0, sr1, sc0, sc1):
    c = lax.axis_index("c")
    s = lax.axis_index("s")
    wid = s * _NC + c

    zero16 = jnp.zeros((16,), jnp.float32)

    def zrows(i, carry):
        rows1[i // (_H // 16), pl.ds((i % (_H // 16)) * 16, 16)] = zero16
        return carry

    lax.fori_loop(0, _C * (_H // 16), zrows, 0)

    def zacc(k, carry):
        j = s * _ZPS + k
        pltpu.sync_copy(rows1, acc.at[pl.ds(j * _C, _C)])
        return carry

    lax.fori_loop(0, _ZPS, zacc, 0)
    plsc.subcore_barrier()

    def ebody(j, carry):
        pltpu.sync_copy(row_hbm.at[wid, j], ridx0)
        pltpu.sync_copy(col_hbm.at[wid, j], cidx0)
        pltpu.async_copy(y_hbm.at[ridx0], rows0, sg0).wait()
        pltpu.sync_copy(rows0, acc.at[cidx0], add=True)
        return carry

    lax.fori_loop(0, _NCHUNK, ebody, 0)
    plsc.subcore_barrier()

    def wout(k, carry):
        j = s * _ZPS + k
        pltpu.sync_copy(acc.at[pl.ds(j * _C, _C)], rows0)
        pltpu.sync_copy(rows0, out_hbm.at[c, pl.ds(j * _C, _C)])
        return carry

    lax.fori_loop(0, _ZPS, wout, 0)


def _seg_sum_sc(y, row3, col3):
    mesh = plsc.VectorSubcoreMesh(
        core_axis_name="c", subcore_axis_name="s",
        num_cores=_NC, num_subcores=_NS)
    f = functools.partial(
        pl.kernel,
        mesh=mesh,
        out_type=jax.ShapeDtypeStruct((_NC, _NPAD, _H), jnp.float32),
        scratch_types=[
            pltpu.VMEM((_C,), jnp.int32),
            pltpu.VMEM((_C,), jnp.int32),
            pltpu.VMEM((_C,), jnp.int32),
            pltpu.VMEM((_C,), jnp.int32),
            pltpu.VMEM((_C, _H), jnp.float32),
            pltpu.VMEM((_C, _H), jnp.float32),
            pltpu.VMEM_SHARED((_NPAD, _H), jnp.float32),
            pltpu.SemaphoreType.DMA,
            pltpu.SemaphoreType.DMA,
            pltpu.SemaphoreType.DMA,
            pltpu.SemaphoreType.DMA,
            pltpu.SemaphoreType.DMA,
            pltpu.SemaphoreType.DMA,
        ],
    )(_seg_sum_body)
    return f(y, row3, col3)


def _gru_head_body(p_ref, x_ref, h_ref, wih_ref, whh_ref, bih_ref, bhh_ref,
                   wg_ref, bg_ref, wax_ref, wag_ref, ba_ref, a_ref, hn_ref):
    xt = p_ref[0] + p_ref[1]
    h0 = h_ref[...]
    gi = jnp.dot(xt, wih_ref[...], preferred_element_type=jnp.float32) + bih_ref[...]
    gh = jnp.dot(h0, whh_ref[...], preferred_element_type=jnp.float32) + bhh_ref[...]
    r = jax.nn.sigmoid(gi[:, :_H] + gh[:, :_H])
    z = jax.nn.sigmoid(gi[:, _H:2 * _H] + gh[:, _H:2 * _H])
    n = jnp.tanh(gi[:, 2 * _H:] + r * gh[:, 2 * _H:])
    hn = (1.0 - z) * n + z * h0
    hn_ref[...] = hn
    g = jnp.maximum(
        jnp.dot(hn, wg_ref[...], preferred_element_type=jnp.float32) + bg_ref[...], 0.0)
    sacc = (jnp.dot(x_ref[...], wax_ref[...], preferred_element_type=jnp.float32)
            + jnp.dot(g, wag_ref[...], preferred_element_type=jnp.float32)
            + ba_ref[...])
    a_ref[...] = jax.nn.softplus(sacc)


def _gru_head(p, x, h, wiht, whht, bih, bhh, wgt, bg, waxt, wagt, ba):
    return pl.pallas_call(
        _gru_head_body,
        grid=(_N // _BLK,),
        in_specs=[
            pl.BlockSpec((_NC, _BLK, _H), lambda i: (0, i, 0)),
            pl.BlockSpec((_BLK, _H), lambda i: (i, 0)),
            pl.BlockSpec((_BLK, _H), lambda i: (i, 0)),
            pl.BlockSpec((_H, 3 * _H), lambda i: (0, 0)),
            pl.BlockSpec((_H, 3 * _H), lambda i: (0, 0)),
            pl.BlockSpec((1, 3 * _H), lambda i: (0, 0)),
            pl.BlockSpec((1, 3 * _H), lambda i: (0, 0)),
            pl.BlockSpec((_H, _H), lambda i: (0, 0)),
            pl.BlockSpec((1, _H), lambda i: (0, 0)),
            pl.BlockSpec((_H, 1), lambda i: (0, 0)),
            pl.BlockSpec((_H, 1), lambda i: (0, 0)),
            pl.BlockSpec((1, 1), lambda i: (0, 0)),
        ],
        out_specs=[
            pl.BlockSpec((_BLK, 1), lambda i: (i, 0)),
            pl.BlockSpec((_BLK, _H), lambda i: (i, 0)),
        ],
        out_shape=[
            jax.ShapeDtypeStruct((_N, 1), jnp.float32),
            jax.ShapeDtypeStruct((_N, _H), jnp.float32),
        ],
    )(p, x, h, wiht, whht, bih, bhh, wgt, bg, waxt, wagt, ba)


def kernel(x, edge_index, h, W1, b1, W_ih, W_hh, b_ih, b_hh, Wg, bg, Wa, ba):
    npad = _EPAD - _E
    row3 = jnp.concatenate(
        [edge_index[0], jnp.zeros((npad,), jnp.int32)]).reshape(_NW, _NCHUNK, _C)
    col3 = jnp.concatenate(
        [edge_index[1], jnp.full((npad,), _N + 64, jnp.int32)]).reshape(_NW, _NCHUNK, _C)
    y = _node_mlp(x, W1.T, b1.reshape(1, _H))
    p = _seg_sum_sc(y, row3, col3)
    a, h_new = _gru_head(
        p, x, h,
        W_ih.T, W_hh.T, b_ih.reshape(1, 3 * _H), b_hh.reshape(1, 3 * _H),
        Wg.T, bg.reshape(1, _H),
        Wa[:, :_H].T, Wa[:, _H:].T, ba.reshape(1, 1))
    return (a, h_new)


# speedup vs baseline: 2.1413x; 2.1413x over previous
import functools

import jax
import jax.numpy as jnp
from jax import lax
from jax.experimental import pallas as pl
from jax.experimental.pallas import tpu as pltpu
from jax.experimental.pallas import tpu_sc as plsc

_N = 10000
_E = 320000
_H = 128

_NC = 2
_NS = 16
_NW = _NC * _NS
_C = 128
_NCHUNK = 80
_EPW = _NCHUNK * _C
_EPAD = _NW * _EPW
_NPAD = 10240
_ZPS = (_NPAD // _C) // _NS

_BLK = 1000


def _mlp_body(x_ref, w_ref, b_ref, y_ref):
    y = jnp.dot(x_ref[...], w_ref[...], preferred_element_type=jnp.float32)
    y_ref[...] = jnp.maximum(y + b_ref[...], 0.0)


def _node_mlp(x, w1t, b1):
    return pl.pallas_call(
        _mlp_body,
        grid=(_N // _BLK,),
        in_specs=[
            pl.BlockSpec((_BLK, _H), lambda i: (i, 0)),
            pl.BlockSpec((_H, _H), lambda i: (0, 0)),
            pl.BlockSpec((1, _H), lambda i: (0, 0)),
        ],
        out_specs=pl.BlockSpec((_BLK, _H), lambda i: (i, 0)),
        out_shape=jax.ShapeDtypeStruct((_N, _H), jnp.float32),
    )(x, w1t, b1)


def _seg_sum_body(y_hbm, row_hbm, col_hbm, out_hbm,
                  ridx0, ridx1, cidx0, cidx1, rows0, rows1, acc,
                  sg0, sg1, sr0, sr1, sc0, sc1):
    c = lax.axis_index("c")
    s = lax.axis_index("s")
    wid = s * _NC + c

    zero16 = jnp.zeros((16,), jnp.float32)

    def zrows(i, carry):
        rows1[i // (_H // 16), pl.ds((i % (_H // 16)) * 16, 16)] = zero16
        return carry

    lax.fori_loop(0, _C * (_H // 16), zrows, 0)

    def zacc(k, carry):
        j = s * _ZPS + k
        pltpu.sync_copy(rows1, acc.at[pl.ds(j * _C, _C)])
        return carry

    lax.fori_loop(0, _ZPS, zacc, 0)
    plsc.subcore_barrier()

    def ebody(j, carry):
        pltpu.sync_copy(row_hbm.at[wid, j], ridx0)
        pltpu.sync_copy(col_hbm.at[wid, j], cidx0)
        pltpu.async_copy(y_hbm.at[ridx0], rows0, sg0).wait()
        pltpu.sync_copy(rows0, acc.at[cidx0], add=True)
        return carry

    lax.fori_loop(0, _NCHUNK, ebody, 0)
    plsc.subcore_barrier()

    def wout(k, carry):
        j = s * _ZPS + k
        pltpu.sync_copy(acc.at[pl.ds(j * _C, _C)], rows0)
        pltpu.sync_copy(rows0, out_hbm.at[c, pl.ds(j * _C, _C)])
        return carry

    lax.fori_loop(0, _ZPS, wout, 0)


def _seg_sum_sc(y, row3, col3):
    mesh = plsc.VectorSubcoreMesh(
        core_axis_name="c", subcore_axis_name="s",
        num_cores=_NC, num_subcores=_NS)
    f = functools.partial(
        pl.kernel,
        mesh=mesh,
        out_type=jax.ShapeDtypeStruct((_NC, _NPAD, _H), jnp.float32),
        scratch_types=[
            pltpu.VMEM((_C,), jnp.int32),
            pltpu.VMEM((_C,), jnp.int32),
            pltpu.VMEM((_C,), jnp.int32),
            pltpu.VMEM((_C,), jnp.int32),
            pltpu.VMEM((_C, _H), jnp.float32),
            pltpu.VMEM((_C, _H), jnp.float32),
            pltpu.VMEM_SHARED((_NPAD, _H), jnp.float32),
            pltpu.SemaphoreType.DMA,
            pltpu.SemaphoreType.DMA,
            pltpu.SemaphoreType.DMA,
            pltpu.SemaphoreType.DMA,
            pltpu.SemaphoreType.DMA,
            pltpu.SemaphoreType.DMA,
        ],
    )(_seg_sum_body)
    return f(y, row3, col3)


def _gru_head_body(p_ref, x_ref, h_ref, wih_ref, whh_ref, bih_ref, bhh_ref,
                   wg_ref, bg_ref, wax_ref, wag_ref, ba_ref, a_ref, hn_ref):
    xt = p_ref[0] + p_ref[1]
    h0 = h_ref[...]
    gi = jnp.dot(xt, wih_ref[...], preferred_element_type=jnp.float32) + bih_ref[...]
    gh = jnp.dot(h0, whh_ref[...], preferred_element_type=jnp.float32) + bhh_ref[...]
    r = jax.nn.sigmoid(gi[:, :_H] + gh[:, :_H])
    z = jax.nn.sigmoid(gi[:, _H:2 * _H] + gh[:, _H:2 * _H])
    n = jnp.tanh(gi[:, 2 * _H:] + r * gh[:, 2 * _H:])
    hn = (1.0 - z) * n + z * h0
    hn_ref[...] = hn
    g = jnp.maximum(
        jnp.dot(hn, wg_ref[...], preferred_element_type=jnp.float32) + bg_ref[...], 0.0)
    sacc = (jnp.dot(x_ref[...], wax_ref[...], preferred_element_type=jnp.float32)
            + jnp.dot(g, wag_ref[...], preferred_element_type=jnp.float32)
            + ba_ref[...])
    a_ref[...] = jax.nn.softplus(sacc)


def _gru_head(p, x, h, wiht, whht, bih, bhh, wgt, bg, waxt, wagt, ba):
    return pl.pallas_call(
        _gru_head_body,
        grid=(_N // _BLK,),
        in_specs=[
            pl.BlockSpec((_NC, _BLK, _H), lambda i: (0, i, 0)),
            pl.BlockSpec((_BLK, _H), lambda i: (i, 0)),
            pl.BlockSpec((_BLK, _H), lambda i: (i, 0)),
            pl.BlockSpec((_H, 3 * _H), lambda i: (0, 0)),
            pl.BlockSpec((_H, 3 * _H), lambda i: (0, 0)),
            pl.BlockSpec((1, 3 * _H), lambda i: (0, 0)),
            pl.BlockSpec((1, 3 * _H), lambda i: (0, 0)),
            pl.BlockSpec((_H, _H), lambda i: (0, 0)),
            pl.BlockSpec((1, _H), lambda i: (0, 0)),
            pl.BlockSpec((_H, 1), lambda i: (0, 0)),
            pl.BlockSpec((_H, 1), lambda i: (0, 0)),
            pl.BlockSpec((1, 1), lambda i: (0, 0)),
        ],
        out_specs=[
            pl.BlockSpec((_BLK, 1), lambda i: (i, 0)),
            pl.BlockSpec((_BLK, _H), lambda i: (i, 0)),
        ],
        out_shape=[
            jax.ShapeDtypeStruct((_N, 1), jnp.float32),
            jax.ShapeDtypeStruct((_N, _H), jnp.float32),
        ],
    )(p, x, h, wiht, whht, bih, bhh, wgt, bg, waxt, wagt, ba)


def kernel(x, edge_index, h, W1, b1, W_ih, W_hh, b_ih, b_hh, Wg, bg, Wa, ba):
    npad = _EPAD - _E
    pad_col = _N + jax.lax.iota(jnp.int32, npad) % (_NPAD - _N)
    pad_row = jax.lax.iota(jnp.int32, npad) % _N
    row3 = jnp.concatenate([edge_index[0], pad_row]).reshape(_NW, _NCHUNK, _C)
    col3 = jnp.concatenate([edge_index[1], pad_col]).reshape(_NW, _NCHUNK, _C)
    y = _node_mlp(x, W1.T, b1.reshape(1, _H))
    p = _seg_sum_sc(y, row3, col3)
    a, h_new = _gru_head(
        p, x, h,
        W_ih.T, W_hh.T, b_ih.reshape(1, 3 * _H), b_hh.reshape(1, 3 * _H),
        Wg.T, bg.reshape(1, _H),
        Wa[:, :_H].T, Wa[:, _H:].T, ba.reshape(1, 1))
    return (a, h_new)


# speedup vs baseline: 3.4633x; 1.6173x over previous
import functools

import jax
import jax.numpy as jnp
from jax import lax
from jax.experimental import pallas as pl
from jax.experimental.pallas import tpu as pltpu
from jax.experimental.pallas import tpu_sc as plsc

_N = 10000
_E = 320000
_H = 128

_NC = 2
_NS = 16
_NW = _NC * _NS
_C = 128
_NCHUNK = 80
_EPW = _NCHUNK * _C
_EPAD = _NW * _EPW
_NPAD = 10240
_ZPS = (_NPAD // _C) // _NS

_BLK = 1000


def _mlp_body(x_ref, w_ref, b_ref, y_ref):
    y = jnp.dot(x_ref[...], w_ref[...], preferred_element_type=jnp.float32)
    y_ref[...] = jnp.maximum(y + b_ref[...], 0.0)


def _node_mlp(x, w1t, b1):
    return pl.pallas_call(
        _mlp_body,
        grid=(_N // _BLK,),
        in_specs=[
            pl.BlockSpec((_BLK, _H), lambda i: (i, 0)),
            pl.BlockSpec((_H, _H), lambda i: (0, 0)),
            pl.BlockSpec((1, _H), lambda i: (0, 0)),
        ],
        out_specs=pl.BlockSpec((_BLK, _H), lambda i: (i, 0)),
        out_shape=jax.ShapeDtypeStruct((_N, _H), jnp.float32),
    )(x, w1t, b1)


def _seg_sum_body(y_hbm, row_hbm, col_hbm, out_hbm,
                  ridx0, ridx1, cidx0, cidx1, rows0, rows1, acc,
                  sg0, sg1, sr0, sr1, sc0, sc1):
    c = lax.axis_index("c")
    s = lax.axis_index("s")
    wid = s * _NC + c

    zero16 = jnp.zeros((16,), jnp.float32)

    def zrows(i, carry):
        rows1[i // (_H // 16), pl.ds((i % (_H // 16)) * 16, 16)] = zero16
        return carry

    lax.fori_loop(0, _C * (_H // 16), zrows, 0)

    def zacc(k, carry):
        j = s * _ZPS + k
        pltpu.sync_copy(rows1, acc.at[pl.ds(j * _C, _C)])
        return carry

    pltpu.async_copy(row_hbm.at[wid, 0], ridx0, sr0)
    pltpu.async_copy(col_hbm.at[wid, 0], cidx0, sc0)
    pltpu.async_copy(row_hbm.at[wid, 1], ridx1, sr1)
    pltpu.async_copy(col_hbm.at[wid, 1], cidx1, sc1)

    lax.fori_loop(0, _ZPS, zacc, 0)
    pltpu.make_async_copy(row_hbm.at[wid, 0], ridx0, sr0).wait()
    pltpu.async_copy(y_hbm.at[ridx0], rows0, sg0)
    plsc.subcore_barrier()

    def pair(jp, carry):
        j0 = jp * 2
        pltpu.make_async_copy(row_hbm.at[wid, 0], ridx1, sr1).wait()
        pltpu.make_async_copy(y_hbm.at[ridx0], rows0, sg0).wait()
        pltpu.async_copy(y_hbm.at[ridx1], rows1, sg1)
        pltpu.make_async_copy(col_hbm.at[wid, 0], cidx0, sc0).wait()
        pltpu.sync_copy(rows0, acc.at[cidx0], add=True)

        @pl.when(j0 + 2 < _NCHUNK)
        def _():
            pltpu.async_copy(row_hbm.at[wid, j0 + 2], ridx0, sr0)
            pltpu.async_copy(col_hbm.at[wid, j0 + 2], cidx0, sc0)

        pltpu.make_async_copy(y_hbm.at[ridx1], rows1, sg1).wait()

        @pl.when(j0 + 2 < _NCHUNK)
        def _():
            pltpu.make_async_copy(row_hbm.at[wid, 0], ridx0, sr0).wait()
            pltpu.async_copy(y_hbm.at[ridx0], rows0, sg0)

        pltpu.make_async_copy(col_hbm.at[wid, 0], cidx1, sc1).wait()
        pltpu.sync_copy(rows1, acc.at[cidx1], add=True)

        @pl.when(j0 + 3 < _NCHUNK)
        def _():
            pltpu.async_copy(row_hbm.at[wid, j0 + 3], ridx1, sr1)
            pltpu.async_copy(col_hbm.at[wid, j0 + 3], cidx1, sc1)

        return carry

    lax.fori_loop(0, _NCHUNK // 2, pair, 0)
    plsc.subcore_barrier()

    def wout(k, carry):
        j = s * _ZPS + k
        pltpu.sync_copy(acc.at[pl.ds(j * _C, _C)], rows0)
        pltpu.sync_copy(rows0, out_hbm.at[c, pl.ds(j * _C, _C)])
        return carry

    lax.fori_loop(0, _ZPS, wout, 0)


def _seg_sum_sc(y, row3, col3):
    mesh = plsc.VectorSubcoreMesh(
        core_axis_name="c", subcore_axis_name="s",
        num_cores=_NC, num_subcores=_NS)
    f = functools.partial(
        pl.kernel,
        mesh=mesh,
        out_type=jax.ShapeDtypeStruct((_NC, _NPAD, _H), jnp.float32),
        scratch_types=[
            pltpu.VMEM((_C,), jnp.int32),
            pltpu.VMEM((_C,), jnp.int32),
            pltpu.VMEM((_C,), jnp.int32),
            pltpu.VMEM((_C,), jnp.int32),
            pltpu.VMEM((_C, _H), jnp.float32),
            pltpu.VMEM((_C, _H), jnp.float32),
            pltpu.VMEM_SHARED((_NPAD, _H), jnp.float32),
            pltpu.SemaphoreType.DMA,
            pltpu.SemaphoreType.DMA,
            pltpu.SemaphoreType.DMA,
            pltpu.SemaphoreType.DMA,
            pltpu.SemaphoreType.DMA,
            pltpu.SemaphoreType.DMA,
        ],
    )(_seg_sum_body)
    return f(y, row3, col3)


def _gru_head_body(p_ref, x_ref, h_ref, wih_ref, whh_ref, bih_ref, bhh_ref,
                   wg_ref, bg_ref, wax_ref, wag_ref, ba_ref, a_ref, hn_ref):
    xt = p_ref[0] + p_ref[1]
    h0 = h_ref[...]
    gi = jnp.dot(xt, wih_ref[...], preferred_element_type=jnp.float32) + bih_ref[...]
    gh = jnp.dot(h0, whh_ref[...], preferred_element_type=jnp.float32) + bhh_ref[...]
    r = jax.nn.sigmoid(gi[:, :_H] + gh[:, :_H])
    z = jax.nn.sigmoid(gi[:, _H:2 * _H] + gh[:, _H:2 * _H])
    n = jnp.tanh(gi[:, 2 * _H:] + r * gh[:, 2 * _H:])
    hn = (1.0 - z) * n + z * h0
    hn_ref[...] = hn
    g = jnp.maximum(
        jnp.dot(hn, wg_ref[...], preferred_element_type=jnp.float32) + bg_ref[...], 0.0)
    sacc = (jnp.dot(x_ref[...], wax_ref[...], preferred_element_type=jnp.float32)
            + jnp.dot(g, wag_ref[...], preferred_element_type=jnp.float32)
            + ba_ref[...])
    a_ref[...] = jax.nn.softplus(sacc)


def _gru_head(p, x, h, wiht, whht, bih, bhh, wgt, bg, waxt, wagt, ba):
    return pl.pallas_call(
        _gru_head_body,
        grid=(_N // _BLK,),
        in_specs=[
            pl.BlockSpec((_NC, _BLK, _H), lambda i: (0, i, 0)),
            pl.BlockSpec((_BLK, _H), lambda i: (i, 0)),
            pl.BlockSpec((_BLK, _H), lambda i: (i, 0)),
            pl.BlockSpec((_H, 3 * _H), lambda i: (0, 0)),
            pl.BlockSpec((_H, 3 * _H), lambda i: (0, 0)),
            pl.BlockSpec((1, 3 * _H), lambda i: (0, 0)),
            pl.BlockSpec((1, 3 * _H), lambda i: (0, 0)),
            pl.BlockSpec((_H, _H), lambda i: (0, 0)),
            pl.BlockSpec((1, _H), lambda i: (0, 0)),
            pl.BlockSpec((_H, 1), lambda i: (0, 0)),
            pl.BlockSpec((_H, 1), lambda i: (0, 0)),
            pl.BlockSpec((1, 1), lambda i: (0, 0)),
        ],
        out_specs=[
            pl.BlockSpec((_BLK, 1), lambda i: (i, 0)),
            pl.BlockSpec((_BLK, _H), lambda i: (i, 0)),
        ],
        out_shape=[
            jax.ShapeDtypeStruct((_N, 1), jnp.float32),
            jax.ShapeDtypeStruct((_N, _H), jnp.float32),
        ],
    )(p, x, h, wiht, whht, bih, bhh, wgt, bg, waxt, wagt, ba)


def kernel(x, edge_index, h, W1, b1, W_ih, W_hh, b_ih, b_hh, Wg, bg, Wa, ba):
    npad = _EPAD - _E
    pad_col = _N + jax.lax.iota(jnp.int32, npad) % (_NPAD - _N)
    pad_row = jax.lax.iota(jnp.int32, npad) % _N
    row3 = jnp.concatenate([edge_index[0], pad_row]).reshape(_NW, _NCHUNK, _C)
    col3 = jnp.concatenate([edge_index[1], pad_col]).reshape(_NW, _NCHUNK, _C)
    y = _node_mlp(x, W1.T, b1.reshape(1, _H))
    p = _seg_sum_sc(y, row3, col3)
    a, h_new = _gru_head(
        p, x, h,
        W_ih.T, W_hh.T, b_ih.reshape(1, 3 * _H), b_hh.reshape(1, 3 * _H),
        Wg.T, bg.reshape(1, _H),
        Wa[:, :_H].T, Wa[:, _H:].T, ba.reshape(1, 1))
    return (a, h_new)


# speedup vs baseline: 3.4639x; 1.0002x over previous
import functools

import jax
import jax.numpy as jnp
from jax import lax
from jax.experimental import pallas as pl
from jax.experimental.pallas import tpu as pltpu
from jax.experimental.pallas import tpu_sc as plsc

_N = 10000
_E = 320000
_H = 128

_NC = 2
_NS = 16
_NW = _NC * _NS
_C = 128
_NCHUNK = 80
_EPW = _NCHUNK * _C
_EPAD = _NW * _EPW
_NPAD = 10240
_ZPS = (_NPAD // _C) // _NS

_BLK = 1000


def _mlp_body(x_ref, w_ref, b_ref, y_ref):
    y = jnp.dot(x_ref[...], w_ref[...], preferred_element_type=jnp.float32)
    y_ref[...] = jnp.maximum(y + b_ref[...], 0.0)


def _node_mlp(x, w1t, b1):
    return pl.pallas_call(
        _mlp_body,
        grid=(_N // _BLK,),
        in_specs=[
            pl.BlockSpec((_BLK, _H), lambda i: (i, 0)),
            pl.BlockSpec((_H, _H), lambda i: (0, 0)),
            pl.BlockSpec((1, _H), lambda i: (0, 0)),
        ],
        out_specs=pl.BlockSpec((_BLK, _H), lambda i: (i, 0)),
        out_shape=jax.ShapeDtypeStruct((_N, _H), jnp.float32),
    )(x, w1t, b1)


def _seg_sum_body(y_hbm, row_hbm, col_hbm, out_hbm,
                  ridx0, ridx1, cidx0, cidx1, cidx2, cidx3, rows0, rows1, acc,
                  sg0, sg1, sr0, sr1, sc0, sc1, sc2, sc3, ss0, ss1):
    c = lax.axis_index("c")
    s = lax.axis_index("s")
    wid = s * _NC + c
    ridx = [ridx0, ridx1]
    rows = [rows0, rows1]
    cidx = [cidx0, cidx1, cidx2, cidx3]
    sgs = [sg0, sg1]
    srs = [sr0, sr1]
    scs = [sc0, sc1, sc2, sc3]
    sss = [ss0, ss1]

    zero16 = jnp.zeros((16,), jnp.float32)

    def zrows(i, carry):
        rows1[i // (_H // 16), pl.ds((i % (_H // 16)) * 16, 16)] = zero16
        return carry

    lax.fori_loop(0, _C * (_H // 16), zrows, 0)

    def zacc(k, carry):
        j = s * _ZPS + k
        pltpu.sync_copy(rows1, acc.at[pl.ds(j * _C, _C)])
        return carry

    pltpu.async_copy(row_hbm.at[wid, 0], ridx0, sr0)
    pltpu.async_copy(col_hbm.at[wid, 0], cidx0, sc0)
    pltpu.async_copy(row_hbm.at[wid, 1], ridx1, sr1)
    pltpu.async_copy(col_hbm.at[wid, 1], cidx1, sc1)

    lax.fori_loop(0, _ZPS, zacc, 0)
    pltpu.make_async_copy(row_hbm.at[wid, 0], ridx0, sr0).wait()
    pltpu.async_copy(y_hbm.at[ridx0], rows0, sg0)
    plsc.subcore_barrier()

    def _chunk(j, m):
        a = m % 2
        b = (m + 1) % 2

        @pl.when(j + 1 < _NCHUNK)
        def _():
            pltpu.make_async_copy(row_hbm.at[wid, 0], ridx[b], srs[b]).wait()

        pltpu.make_async_copy(y_hbm.at[ridx[a]], rows[a], sgs[a]).wait()

        @pl.when(j >= 2)
        def _():
            pltpu.make_async_copy(rows[b], acc.at[cidx[(m + 3) % 4]], sss[b]).wait()

        @pl.when(j + 1 < _NCHUNK)
        def _():
            pltpu.async_copy(y_hbm.at[ridx[b]], rows[b], sgs[b])

        pltpu.make_async_copy(col_hbm.at[wid, 0], cidx[m], scs[m]).wait()
        pltpu.async_copy(rows[a], acc.at[cidx[m]], sss[a], add=True)

        @pl.when(j + 2 < _NCHUNK)
        def _():
            pltpu.async_copy(row_hbm.at[wid, j + 2], ridx[a], srs[a])
            pltpu.async_copy(col_hbm.at[wid, j + 2], cidx[(m + 2) % 4], scs[(m + 2) % 4])

    def quad(q, carry):
        j0 = q * 4
        for m in range(4):
            _chunk(j0 + m, m)
        return carry

    lax.fori_loop(0, _NCHUNK // 4, quad, 0)
    pltpu.make_async_copy(rows[0], acc.at[cidx[2]], sss[0]).wait()
    pltpu.make_async_copy(rows[1], acc.at[cidx[3]], sss[1]).wait()
    plsc.subcore_barrier()

    def wout(k, carry):
        j = s * _ZPS + k
        pltpu.sync_copy(acc.at[pl.ds(j * _C, _C)], rows0)
        pltpu.sync_copy(rows0, out_hbm.at[c, pl.ds(j * _C, _C)])
        return carry

    lax.fori_loop(0, _ZPS, wout, 0)


def _seg_sum_sc(y, row3, col3):
    mesh = plsc.VectorSubcoreMesh(
        core_axis_name="c", subcore_axis_name="s",
        num_cores=_NC, num_subcores=_NS)
    f = functools.partial(
        pl.kernel,
        mesh=mesh,
        out_type=jax.ShapeDtypeStruct((_NC, _NPAD, _H), jnp.float32),
        scratch_types=(
            [pltpu.VMEM((_C,), jnp.int32)] * 6
            + [pltpu.VMEM((_C, _H), jnp.float32)] * 2
            + [pltpu.VMEM_SHARED((_NPAD, _H), jnp.float32)]
            + [pltpu.SemaphoreType.DMA] * 10
        ),
    )(_seg_sum_body)
    return f(y, row3, col3)


def _gru_head_body(p_ref, x_ref, h_ref, wih_ref, whh_ref, bih_ref, bhh_ref,
                   wg_ref, bg_ref, wax_ref, wag_ref, ba_ref, a_ref, hn_ref):
    xt = p_ref[0] + p_ref[1]
    h0 = h_ref[...]
    gi = jnp.dot(xt, wih_ref[...], preferred_element_type=jnp.float32) + bih_ref[...]
    gh = jnp.dot(h0, whh_ref[...], preferred_element_type=jnp.float32) + bhh_ref[...]
    r = jax.nn.sigmoid(gi[:, :_H] + gh[:, :_H])
    z = jax.nn.sigmoid(gi[:, _H:2 * _H] + gh[:, _H:2 * _H])
    n = jnp.tanh(gi[:, 2 * _H:] + r * gh[:, 2 * _H:])
    hn = (1.0 - z) * n + z * h0
    hn_ref[...] = hn
    g = jnp.maximum(
        jnp.dot(hn, wg_ref[...], preferred_element_type=jnp.float32) + bg_ref[...], 0.0)
    sacc = (jnp.dot(x_ref[...], wax_ref[...], preferred_element_type=jnp.float32)
            + jnp.dot(g, wag_ref[...], preferred_element_type=jnp.float32)
            + ba_ref[...])
    a_ref[...] = jax.nn.softplus(sacc)


def _gru_head(p, x, h, wiht, whht, bih, bhh, wgt, bg, waxt, wagt, ba):
    return pl.pallas_call(
        _gru_head_body,
        grid=(_N // _BLK,),
        in_specs=[
            pl.BlockSpec((_NC, _BLK, _H), lambda i: (0, i, 0)),
            pl.BlockSpec((_BLK, _H), lambda i: (i, 0)),
            pl.BlockSpec((_BLK, _H), lambda i: (i, 0)),
            pl.BlockSpec((_H, 3 * _H), lambda i: (0, 0)),
            pl.BlockSpec((_H, 3 * _H), lambda i: (0, 0)),
            pl.BlockSpec((1, 3 * _H), lambda i: (0, 0)),
            pl.BlockSpec((1, 3 * _H), lambda i: (0, 0)),
            pl.BlockSpec((_H, _H), lambda i: (0, 0)),
            pl.BlockSpec((1, _H), lambda i: (0, 0)),
            pl.BlockSpec((_H, 1), lambda i: (0, 0)),
            pl.BlockSpec((_H, 1), lambda i: (0, 0)),
            pl.BlockSpec((1, 1), lambda i: (0, 0)),
        ],
        out_specs=[
            pl.BlockSpec((_BLK, 1), lambda i: (i, 0)),
            pl.BlockSpec((_BLK, _H), lambda i: (i, 0)),
        ],
        out_shape=[
            jax.ShapeDtypeStruct((_N, 1), jnp.float32),
            jax.ShapeDtypeStruct((_N, _H), jnp.float32),
        ],
    )(p, x, h, wiht, whht, bih, bhh, wgt, bg, waxt, wagt, ba)


def kernel(x, edge_index, h, W1, b1, W_ih, W_hh, b_ih, b_hh, Wg, bg, Wa, ba):
    npad = _EPAD - _E
    pad_col = _N + jax.lax.iota(jnp.int32, npad) % (_NPAD - _N)
    pad_row = jax.lax.iota(jnp.int32, npad) % _N
    row3 = jnp.concatenate([edge_index[0], pad_row]).reshape(_NW, _NCHUNK, _C)
    col3 = jnp.concatenate([edge_index[1], pad_col]).reshape(_NW, _NCHUNK, _C)
    y = _node_mlp(x, W1.T, b1.reshape(1, _H))
    p = _seg_sum_sc(y, row3, col3)
    a, h_new = _gru_head(
        p, x, h,
        W_ih.T, W_hh.T, b_ih.reshape(1, 3 * _H), b_hh.reshape(1, 3 * _H),
        Wg.T, bg.reshape(1, _H),
        Wa[:, :_H].T, Wa[:, _H:].T, ba.reshape(1, 1))
    return (a, h_new)


# speedup vs baseline: 3.4793x; 1.0044x over previous
import functools

import jax
import jax.numpy as jnp
from jax import lax
from jax.experimental import pallas as pl
from jax.experimental.pallas import tpu as pltpu
from jax.experimental.pallas import tpu_sc as plsc

_N = 10000
_E = 320000
_H = 128

_NC = 2
_NS = 16
_NW = _NC * _NS
_C = 128
_NCHUNK = 80
_EPW = _NCHUNK * _C
_EPAD = _NW * _EPW
_NPAD = 10240
_ZPS = (_NPAD // _C) // _NS

_BLK = 1000


def _mlp_body(x_ref, w_ref, b_ref, y_ref):
    y = jnp.dot(x_ref[...], w_ref[...], preferred_element_type=jnp.float32)
    y_ref[...] = jnp.maximum(y + b_ref[...], 0.0)


def _node_mlp(x, w1t, b1):
    return pl.pallas_call(
        _mlp_body,
        grid=(_N // _BLK,),
        in_specs=[
            pl.BlockSpec((_BLK, _H), lambda i: (i, 0)),
            pl.BlockSpec((_H, _H), lambda i: (0, 0)),
            pl.BlockSpec((1, _H), lambda i: (0, 0)),
        ],
        out_specs=pl.BlockSpec((_BLK, _H), lambda i: (i, 0)),
        out_shape=jax.ShapeDtypeStruct((_N, _H), jnp.float32),
    )(x, w1t, b1)


def _seg_sum_body(y_hbm, row_hbm, col_hbm, out_hbm,
                  ridx0, ridx1, cidx0, cidx1, cidx2, cidx3, rows0, rows1, acc,
                  sg0, sg1, sr0, sr1, sc0, sc1, sc2, sc3, ss0, ss1):
    c = lax.axis_index("c")
    s = lax.axis_index("s")
    wid = s * _NC + c
    ridx = [ridx0, ridx1]
    rows = [rows0, rows1]
    cidx = [cidx0, cidx1, cidx2, cidx3]
    sgs = [sg0, sg1]
    srs = [sr0, sr1]
    scs = [sc0, sc1, sc2, sc3]
    sss = [ss0, ss1]

    zero16 = jnp.zeros((16,), jnp.float32)

    def zrows(i, carry):
        rows1[i // (_H // 16), pl.ds((i % (_H // 16)) * 16, 16)] = zero16
        return carry

    lax.fori_loop(0, _C * (_H // 16), zrows, 0)

    def zacc(k, carry):
        j = s * _ZPS + k
        pltpu.sync_copy(rows1, acc.at[pl.ds(j * _C, _C)])
        return carry

    pltpu.async_copy(row_hbm.at[wid, 0], ridx0, sr0)
    pltpu.async_copy(col_hbm.at[wid, 0], cidx0, sc0)
    pltpu.async_copy(row_hbm.at[wid, 1], ridx1, sr1)
    pltpu.async_copy(col_hbm.at[wid, 1], cidx1, sc1)

    lax.fori_loop(0, _ZPS, zacc, 0)
    pltpu.make_async_copy(row_hbm.at[wid, 0], ridx0, sr0).wait()
    pltpu.async_copy(y_hbm.at[ridx0], rows0, sg0)
    plsc.subcore_barrier()

    def _chunk(j, m):
        a = m % 2
        b = (m + 1) % 2

        @pl.when(j + 1 < _NCHUNK)
        def _():
            pltpu.make_async_copy(row_hbm.at[wid, 0], ridx[b], srs[b]).wait()

        pltpu.make_async_copy(y_hbm.at[ridx[a]], rows[a], sgs[a]).wait()

        @pl.when(j >= 2)
        def _():
            pltpu.make_async_copy(rows[b], acc.at[cidx[(m + 3) % 4]], sss[b]).wait()

        @pl.when(j + 1 < _NCHUNK)
        def _():
            pltpu.async_copy(y_hbm.at[ridx[b]], rows[b], sgs[b])

        pltpu.make_async_copy(col_hbm.at[wid, 0], cidx[m], scs[m]).wait()
        pltpu.async_copy(rows[a], acc.at[cidx[m]], sss[a], add=True)

        @pl.when(j + 2 < _NCHUNK)
        def _():
            pltpu.async_copy(row_hbm.at[wid, j + 2], ridx[a], srs[a])
            pltpu.async_copy(col_hbm.at[wid, j + 2], cidx[(m + 2) % 4], scs[(m + 2) % 4])

    def quad(q, carry):
        j0 = q * 4
        for m in range(4):
            _chunk(j0 + m, m)
        return carry

    lax.fori_loop(0, _NCHUNK // 4, quad, 0)
    pltpu.make_async_copy(rows[0], acc.at[cidx[2]], sss[0]).wait()
    pltpu.make_async_copy(rows[1], acc.at[cidx[3]], sss[1]).wait()
    plsc.subcore_barrier()

    def wout(k, carry):
        j = s * _ZPS + k
        pltpu.sync_copy(acc.at[pl.ds(j * _C, _C)], out_hbm.at[c, pl.ds(j * _C, _C)])
        return carry

    lax.fori_loop(0, _ZPS, wout, 0)


def _seg_sum_sc(y, row3, col3):
    mesh = plsc.VectorSubcoreMesh(
        core_axis_name="c", subcore_axis_name="s",
        num_cores=_NC, num_subcores=_NS)
    f = functools.partial(
        pl.kernel,
        mesh=mesh,
        out_type=jax.ShapeDtypeStruct((_NC, _NPAD, _H), jnp.float32),
        scratch_types=(
            [pltpu.VMEM((_C,), jnp.int32)] * 6
            + [pltpu.VMEM((_C, _H), jnp.float32)] * 2
            + [pltpu.VMEM_SHARED((_NPAD, _H), jnp.float32)]
            + [pltpu.SemaphoreType.DMA] * 10
        ),
    )(_seg_sum_body)
    return f(y, row3, col3)


def _gru_head_body(p_ref, x_ref, h_ref, wih_ref, whh_ref, bih_ref, bhh_ref,
                   wg_ref, bg_ref, wax_ref, wag_ref, ba_ref, a_ref, hn_ref):
    xt = p_ref[0] + p_ref[1]
    h0 = h_ref[...]
    gi = jnp.dot(xt, wih_ref[...], preferred_element_type=jnp.float32) + bih_ref[...]
    gh = jnp.dot(h0, whh_ref[...], preferred_element_type=jnp.float32) + bhh_ref[...]
    r = jax.nn.sigmoid(gi[:, :_H] + gh[:, :_H])
    z = jax.nn.sigmoid(gi[:, _H:2 * _H] + gh[:, _H:2 * _H])
    n = jnp.tanh(gi[:, 2 * _H:] + r * gh[:, 2 * _H:])
    hn = (1.0 - z) * n + z * h0
    hn_ref[...] = hn
    g = jnp.maximum(
        jnp.dot(hn, wg_ref[...], preferred_element_type=jnp.float32) + bg_ref[...], 0.0)
    sacc = (jnp.dot(x_ref[...], wax_ref[...], preferred_element_type=jnp.float32)
            + jnp.dot(g, wag_ref[...], preferred_element_type=jnp.float32)
            + ba_ref[...])
    a_ref[...] = jax.nn.softplus(sacc)


def _gru_head(p, x, h, wiht, whht, bih, bhh, wgt, bg, waxt, wagt, ba):
    return pl.pallas_call(
        _gru_head_body,
        grid=(_N // _BLK,),
        in_specs=[
            pl.BlockSpec((_NC, _BLK, _H), lambda i: (0, i, 0)),
            pl.BlockSpec((_BLK, _H), lambda i: (i, 0)),
            pl.BlockSpec((_BLK, _H), lambda i: (i, 0)),
            pl.BlockSpec((_H, 3 * _H), lambda i: (0, 0)),
            pl.BlockSpec((_H, 3 * _H), lambda i: (0, 0)),
            pl.BlockSpec((1, 3 * _H), lambda i: (0, 0)),
            pl.BlockSpec((1, 3 * _H), lambda i: (0, 0)),
            pl.BlockSpec((_H, _H), lambda i: (0, 0)),
            pl.BlockSpec((1, _H), lambda i: (0, 0)),
            pl.BlockSpec((_H, 1), lambda i: (0, 0)),
            pl.BlockSpec((_H, 1), lambda i: (0, 0)),
            pl.BlockSpec((1, 1), lambda i: (0, 0)),
        ],
        out_specs=[
            pl.BlockSpec((_BLK, 1), lambda i: (i, 0)),
            pl.BlockSpec((_BLK, _H), lambda i: (i, 0)),
        ],
        out_shape=[
            jax.ShapeDtypeStruct((_N, 1), jnp.float32),
            jax.ShapeDtypeStruct((_N, _H), jnp.float32),
        ],
    )(p, x, h, wiht, whht, bih, bhh, wgt, bg, waxt, wagt, ba)


def kernel(x, edge_index, h, W1, b1, W_ih, W_hh, b_ih, b_hh, Wg, bg, Wa, ba):
    npad = _EPAD - _E
    pad_col = _N + jax.lax.iota(jnp.int32, npad) % (_NPAD - _N)
    pad_row = jax.lax.iota(jnp.int32, npad) % _N
    row3 = jnp.concatenate([edge_index[0], pad_row]).reshape(_NW, _NCHUNK, _C)
    col3 = jnp.concatenate([edge_index[1], pad_col]).reshape(_NW, _NCHUNK, _C)
    y = _node_mlp(x, W1.T, b1.reshape(1, _H))
    p = _seg_sum_sc(y, row3, col3)
    a, h_new = _gru_head(
        p, x, h,
        W_ih.T, W_hh.T, b_ih.reshape(1, 3 * _H), b_hh.reshape(1, 3 * _H),
        Wg.T, bg.reshape(1, _H),
        Wa[:, :_H].T, Wa[:, _H:].T, ba.reshape(1, 1))
    return (a, h_new)


# speedup vs baseline: 4.1525x; 1.1935x over previous
import functools

import jax
import jax.numpy as jnp
from jax import lax
from jax.experimental import pallas as pl
from jax.experimental.pallas import tpu as pltpu
from jax.experimental.pallas import tpu_sc as plsc

_N = 10000
_E = 320000
_H = 128

_NC = 2
_NS = 16
_NW = _NC * _NS
_C = 96
_NCHUNK = 105
_EPW = _NCHUNK * _C
_EPAD = _NW * _EPW
_NPAD = 10240
_ZC = 64
_ZPS = (_NPAD // _ZC) // _NS
_WC = 128
_WPS = (_NPAD // _WC) // _NS

_BLK = 1000


def _mlp_body(x_ref, w_ref, b_ref, y_ref):
    y = jnp.dot(x_ref[...], w_ref[...], preferred_element_type=jnp.float32)
    y_ref[...] = jnp.maximum(y + b_ref[...], 0.0)


def _node_mlp(x, w1t, b1):
    return pl.pallas_call(
        _mlp_body,
        grid=(_N // _BLK,),
        in_specs=[
            pl.BlockSpec((_BLK, _H), lambda i: (i, 0)),
            pl.BlockSpec((_H, _H), lambda i: (0, 0)),
            pl.BlockSpec((1, _H), lambda i: (0, 0)),
        ],
        out_specs=pl.BlockSpec((_BLK, _H), lambda i: (i, 0)),
        out_shape=jax.ShapeDtypeStruct((_N, _H), jnp.float32),
    )(x, w1t, b1)


def _seg_sum_body(y_hbm, row_hbm, col_hbm, out_hbm,
                  ridx0, ridx1, ridx2, cidx0, cidx1, cidx2,
                  rows0, rows1, rows2, acc,
                  sg0, sg1, sg2, sr0, sr1, sr2, sc0, sc1, sc2,
                  ss0, ss1, ss2):
    c = lax.axis_index("c")
    s = lax.axis_index("s")
    wid = s * _NC + c
    ridx = [ridx0, ridx1, ridx2]
    cidx = [cidx0, cidx1, cidx2]
    rows = [rows0, rows1, rows2]
    sgs = [sg0, sg1, sg2]
    srs = [sr0, sr1, sr2]
    scs = [sc0, sc1, sc2]
    sss = [ss0, ss1, ss2]

    pltpu.async_copy(row_hbm.at[wid, 0], ridx0, sr0)
    pltpu.async_copy(row_hbm.at[wid, 1], ridx1, sr1)
    pltpu.async_copy(row_hbm.at[wid, 2], ridx2, sr2)
    pltpu.async_copy(col_hbm.at[wid, 0], cidx0, sc0)
    pltpu.async_copy(col_hbm.at[wid, 1], cidx1, sc1)

    zero16 = jnp.zeros((16,), jnp.float32)

    def zrows(i, carry):
        rows2[i // (_H // 16), pl.ds((i % (_H // 16)) * 16, 16)] = zero16
        return carry

    lax.fori_loop(0, _C * (_H // 16), zrows, 0)

    def zacc(k, carry):
        j = s * _ZPS + k
        pltpu.sync_copy(rows2.at[pl.ds(0, _ZC)], acc.at[pl.ds(j * _ZC, _ZC)])
        return carry

    lax.fori_loop(0, _ZPS, zacc, 0)

    pltpu.make_async_copy(row_hbm.at[wid, 0], ridx0, sr0).wait()
    pltpu.async_copy(y_hbm.at[ridx0], rows0, sg0)
    pltpu.make_async_copy(row_hbm.at[wid, 0], ridx1, sr1).wait()
    pltpu.async_copy(y_hbm.at[ridx1], rows1, sg1)
    plsc.subcore_barrier()

    def _chunk(j, m):
        n2 = (m + 2) % 3
        pltpu.make_async_copy(y_hbm.at[ridx[m]], rows[m], sgs[m]).wait()

        @pl.when(j >= 1)
        def _():
            pltpu.make_async_copy(rows[n2], acc.at[cidx[n2]], sss[n2]).wait()

        @pl.when(j + 2 < _NCHUNK)
        def _():
            pltpu.make_async_copy(row_hbm.at[wid, 0], ridx[n2], srs[n2]).wait()
            pltpu.async_copy(y_hbm.at[ridx[n2]], rows[n2], sgs[n2])

        pltpu.make_async_copy(col_hbm.at[wid, 0], cidx[m], scs[m]).wait()
        pltpu.async_copy(rows[m], acc.at[cidx[m]], sss[m], add=True)

        @pl.when(j + 3 < _NCHUNK)
        def _():
            pltpu.async_copy(row_hbm.at[wid, j + 3], ridx[m], srs[m])

        @pl.when(j + 2 < _NCHUNK)
        def _():
            pltpu.async_copy(col_hbm.at[wid, j + 2], cidx[n2], scs[n2])

    def triple(q, carry):
        j0 = q * 3
        for m in range(3):
            _chunk(j0 + m, m)
        return carry

    lax.fori_loop(0, _NCHUNK // 3, triple, 0)
    pltpu.make_async_copy(rows[2], acc.at[cidx[2]], sss[2]).wait()
    plsc.subcore_barrier()

    def wout(k, carry):
        j = s * _WPS + k
        pltpu.sync_copy(acc.at[pl.ds(j * _WC, _WC)], out_hbm.at[c, pl.ds(j * _WC, _WC)])
        return carry

    lax.fori_loop(0, _WPS, wout, 0)


def _seg_sum_sc(y, row3, col3):
    mesh = plsc.VectorSubcoreMesh(
        core_axis_name="c", subcore_axis_name="s",
        num_cores=_NC, num_subcores=_NS)
    f = functools.partial(
        pl.kernel,
        mesh=mesh,
        out_type=jax.ShapeDtypeStruct((_NC, _NPAD, _H), jnp.float32),
        scratch_types=(
            [pltpu.VMEM((_C,), jnp.int32)] * 6
            + [pltpu.VMEM((_C, _H), jnp.float32)] * 3
            + [pltpu.VMEM_SHARED((_NPAD, _H), jnp.float32)]
            + [pltpu.SemaphoreType.DMA] * 12
        ),
    )(_seg_sum_body)
    return f(y, row3, col3)


def _gru_head_body(p_ref, x_ref, h_ref, wih_ref, whh_ref, bih_ref, bhh_ref,
                   wg_ref, bg_ref, wax_ref, wag_ref, ba_ref, a_ref, hn_ref):
    xt = p_ref[0] + p_ref[1]
    h0 = h_ref[...]
    gi = jnp.dot(xt, wih_ref[...], preferred_element_type=jnp.float32) + bih_ref[...]
    gh = jnp.dot(h0, whh_ref[...], preferred_element_type=jnp.float32) + bhh_ref[...]
    r = jax.nn.sigmoid(gi[:, :_H] + gh[:, :_H])
    z = jax.nn.sigmoid(gi[:, _H:2 * _H] + gh[:, _H:2 * _H])
    n = jnp.tanh(gi[:, 2 * _H:] + r * gh[:, 2 * _H:])
    hn = (1.0 - z) * n + z * h0
    hn_ref[...] = hn
    g = jnp.maximum(
        jnp.dot(hn, wg_ref[...], preferred_element_type=jnp.float32) + bg_ref[...], 0.0)
    sacc = (jnp.dot(x_ref[...], wax_ref[...], preferred_element_type=jnp.float32)
            + jnp.dot(g, wag_ref[...], preferred_element_type=jnp.float32)
            + ba_ref[...])
    a_ref[...] = jax.nn.softplus(sacc)


def _gru_head(p, x, h, wiht, whht, bih, bhh, wgt, bg, waxt, wagt, ba):
    return pl.pallas_call(
        _gru_head_body,
        grid=(_N // _BLK,),
        in_specs=[
            pl.BlockSpec((_NC, _BLK, _H), lambda i: (0, i, 0)),
            pl.BlockSpec((_BLK, _H), lambda i: (i, 0)),
            pl.BlockSpec((_BLK, _H), lambda i: (i, 0)),
            pl.BlockSpec((_H, 3 * _H), lambda i: (0, 0)),
            pl.BlockSpec((_H, 3 * _H), lambda i: (0, 0)),
            pl.BlockSpec((1, 3 * _H), lambda i: (0, 0)),
            pl.BlockSpec((1, 3 * _H), lambda i: (0, 0)),
            pl.BlockSpec((_H, _H), lambda i: (0, 0)),
            pl.BlockSpec((1, _H), lambda i: (0, 0)),
            pl.BlockSpec((_H, 1), lambda i: (0, 0)),
            pl.BlockSpec((_H, 1), lambda i: (0, 0)),
            pl.BlockSpec((1, 1), lambda i: (0, 0)),
        ],
        out_specs=[
            pl.BlockSpec((_BLK, 1), lambda i: (i, 0)),
            pl.BlockSpec((_BLK, _H), lambda i: (i, 0)),
        ],
        out_shape=[
            jax.ShapeDtypeStruct((_N, 1), jnp.float32),
            jax.ShapeDtypeStruct((_N, _H), jnp.float32),
        ],
    )(p, x, h, wiht, whht, bih, bhh, wgt, bg, waxt, wagt, ba)


def kernel(x, edge_index, h, W1, b1, W_ih, W_hh, b_ih, b_hh, Wg, bg, Wa, ba):
    npad = _EPAD - _E
    pad_col = _N + jax.lax.iota(jnp.int32, npad) % (_NPAD - _N)
    pad_row = jax.lax.iota(jnp.int32, npad) % _N
    row3 = jnp.concatenate([edge_index[0], pad_row]).reshape(_NW, _NCHUNK, _C)
    col3 = jnp.concatenate([edge_index[1], pad_col]).reshape(_NW, _NCHUNK, _C)
    y = _node_mlp(x, W1.T, b1.reshape(1, _H))
    p = _seg_sum_sc(y, row3, col3)
    a, h_new = _gru_head(
        p, x, h,
        W_ih.T, W_hh.T, b_ih.reshape(1, 3 * _H), b_hh.reshape(1, 3 * _H),
        Wg.T, bg.reshape(1, _H),
        Wa[:, :_H].T, Wa[:, _H:].T, ba.reshape(1, 1))
    return (a, h_new)


# speedup vs baseline: 4.1612x; 1.0021x over previous
import functools

import jax
import jax.numpy as jnp
from jax import lax
from jax.experimental import pallas as pl
from jax.experimental.pallas import tpu as pltpu
from jax.experimental.pallas import tpu_sc as plsc

_N = 10000
_E = 320000
_H = 128

_NC = 2
_NS = 16
_NW = _NC * _NS
_C = 80
_NCHUNK = 128
_EPW = _NCHUNK * _C
_EPAD = _NW * _EPW
_NPAD = 10240
_ZC = 64
_ZPS = (_NPAD // _ZC) // _NS
_WC = 128
_WPS = (_NPAD // _WC) // _NS

_BLK = 1000


def _mlp_body(x_ref, w_ref, b_ref, y_ref):
    y = jnp.dot(x_ref[...], w_ref[...], preferred_element_type=jnp.float32)
    y_ref[...] = jnp.maximum(y + b_ref[...], 0.0)


def _node_mlp(x, w1t, b1):
    return pl.pallas_call(
        _mlp_body,
        grid=(_N // _BLK,),
        in_specs=[
            pl.BlockSpec((_BLK, _H), lambda i: (i, 0)),
            pl.BlockSpec((_H, _H), lambda i: (0, 0)),
            pl.BlockSpec((1, _H), lambda i: (0, 0)),
        ],
        out_specs=pl.BlockSpec((_BLK, _H), lambda i: (i, 0)),
        out_shape=jax.ShapeDtypeStruct((_N, _H), jnp.float32),
    )(x, w1t, b1)


def _seg_sum_body(y_hbm, row_hbm, col_hbm, out_hbm,
                  ridx0, ridx1, ridx2, ridx3, cidx0, cidx1, cidx2, cidx3,
                  rows0, rows1, rows2, rows3, acc,
                  sg0, sg1, sg2, sg3, sr0, sr1, sr2, sr3,
                  sc0, sc1, sc2, sc3, ss0, ss1, ss2, ss3):
    c = lax.axis_index("c")
    s = lax.axis_index("s")
    wid = s * _NC + c
    ridx = [ridx0, ridx1, ridx2, ridx3]
    cidx = [cidx0, cidx1, cidx2, cidx3]
    rows = [rows0, rows1, rows2, rows3]
    sgs = [sg0, sg1, sg2, sg3]
    srs = [sr0, sr1, sr2, sr3]
    scs = [sc0, sc1, sc2, sc3]
    sss = [ss0, ss1, ss2, ss3]

    pltpu.async_copy(row_hbm.at[wid, 0], ridx0, sr0)
    pltpu.async_copy(row_hbm.at[wid, 1], ridx1, sr1)
    pltpu.async_copy(row_hbm.at[wid, 2], ridx2, sr2)
    pltpu.async_copy(row_hbm.at[wid, 3], ridx3, sr3)
    pltpu.async_copy(col_hbm.at[wid, 0], cidx0, sc0)
    pltpu.async_copy(col_hbm.at[wid, 1], cidx1, sc1)

    zero16 = jnp.zeros((16,), jnp.float32)

    def zrows(i, carry):
        rows3[i // (_H // 16), pl.ds((i % (_H // 16)) * 16, 16)] = zero16
        return carry

    lax.fori_loop(0, _C * (_H // 16), zrows, 0)

    def zacc(k, carry):
        j = s * _ZPS + k
        pltpu.sync_copy(rows3.at[pl.ds(0, _ZC)], acc.at[pl.ds(j * _ZC, _ZC)])
        return carry

    lax.fori_loop(0, _ZPS, zacc, 0)

    pltpu.make_async_copy(row_hbm.at[wid, 0], ridx0, sr0).wait()
    pltpu.async_copy(y_hbm.at[ridx0], rows0, sg0)
    pltpu.make_async_copy(row_hbm.at[wid, 0], ridx1, sr1).wait()
    pltpu.async_copy(y_hbm.at[ridx1], rows1, sg1)
    pltpu.make_async_copy(row_hbm.at[wid, 0], ridx2, sr2).wait()
    pltpu.async_copy(y_hbm.at[ridx2], rows2, sg2)
    plsc.subcore_barrier()

    def _chunk(j, m):
        n3 = (m + 3) % 4
        pltpu.make_async_copy(y_hbm.at[ridx[m]], rows[m], sgs[m]).wait()

        @pl.when(j >= 1)
        def _():
            pltpu.make_async_copy(rows[n3], acc.at[cidx[n3]], sss[n3]).wait()

        @pl.when(j + 3 < _NCHUNK)
        def _():
            pltpu.make_async_copy(row_hbm.at[wid, 0], ridx[n3], srs[n3]).wait()
            pltpu.async_copy(y_hbm.at[ridx[n3]], rows[n3], sgs[n3])

        pltpu.make_async_copy(col_hbm.at[wid, 0], cidx[m], scs[m]).wait()
        pltpu.async_copy(rows[m], acc.at[cidx[m]], sss[m], add=True)

        @pl.when(j + 4 < _NCHUNK)
        def _():
            pltpu.async_copy(row_hbm.at[wid, j + 4], ridx[m], srs[m])

        @pl.when(j + 2 < _NCHUNK)
        def _():
            pltpu.async_copy(col_hbm.at[wid, j + 2], cidx[(m + 2) % 4], scs[(m + 2) % 4])

    def quad(q, carry):
        j0 = q * 4
        for m in range(4):
            _chunk(j0 + m, m)
        return carry

    lax.fori_loop(0, _NCHUNK // 4, quad, 0)
    pltpu.make_async_copy(rows[3], acc.at[cidx[3]], sss[3]).wait()
    plsc.subcore_barrier()

    def wout(k, carry):
        j = s * _WPS + k
        pltpu.sync_copy(acc.at[pl.ds(j * _WC, _WC)], out_hbm.at[c, pl.ds(j * _WC, _WC)])
        return carry

    lax.fori_loop(0, _WPS, wout, 0)


def _seg_sum_sc(y, row3, col3):
    mesh = plsc.VectorSubcoreMesh(
        core_axis_name="c", subcore_axis_name="s",
        num_cores=_NC, num_subcores=_NS)
    f = functools.partial(
        pl.kernel,
        mesh=mesh,
        out_type=jax.ShapeDtypeStruct((_NC, _NPAD, _H), jnp.float32),
        scratch_types=(
            [pltpu.VMEM((_C,), jnp.int32)] * 8
            + [pltpu.VMEM((_C, _H), jnp.float32)] * 4
            + [pltpu.VMEM_SHARED((_NPAD, _H), jnp.float32)]
            + [pltpu.SemaphoreType.DMA] * 16
        ),
    )(_seg_sum_body)
    return f(y, row3, col3)


def _gru_head_body(p_ref, x_ref, h_ref, wih_ref, whh_ref, bih_ref, bhh_ref,
                   wg_ref, bg_ref, wax_ref, wag_ref, ba_ref, a_ref, hn_ref):
    xt = p_ref[0] + p_ref[1]
    h0 = h_ref[...]
    gi = jnp.dot(xt, wih_ref[...], preferred_element_type=jnp.float32) + bih_ref[...]
    gh = jnp.dot(h0, whh_ref[...], preferred_element_type=jnp.float32) + bhh_ref[...]
    r = jax.nn.sigmoid(gi[:, :_H] + gh[:, :_H])
    z = jax.nn.sigmoid(gi[:, _H:2 * _H] + gh[:, _H:2 * _H])
    n = jnp.tanh(gi[:, 2 * _H:] + r * gh[:, 2 * _H:])
    hn = (1.0 - z) * n + z * h0
    hn_ref[...] = hn
    g = jnp.maximum(
        jnp.dot(hn, wg_ref[...], preferred_element_type=jnp.float32) + bg_ref[...], 0.0)
    sacc = (jnp.dot(x_ref[...], wax_ref[...], preferred_element_type=jnp.float32)
            + jnp.dot(g, wag_ref[...], preferred_element_type=jnp.float32)
            + ba_ref[...])
    a_ref[...] = jax.nn.softplus(sacc)


def _gru_head(p, x, h, wiht, whht, bih, bhh, wgt, bg, waxt, wagt, ba):
    return pl.pallas_call(
        _gru_head_body,
        grid=(_N // _BLK,),
        in_specs=[
            pl.BlockSpec((_NC, _BLK, _H), lambda i: (0, i, 0)),
            pl.BlockSpec((_BLK, _H), lambda i: (i, 0)),
            pl.BlockSpec((_BLK, _H), lambda i: (i, 0)),
            pl.BlockSpec((_H, 3 * _H), lambda i: (0, 0)),
            pl.BlockSpec((_H, 3 * _H), lambda i: (0, 0)),
            pl.BlockSpec((1, 3 * _H), lambda i: (0, 0)),
            pl.BlockSpec((1, 3 * _H), lambda i: (0, 0)),
            pl.BlockSpec((_H, _H), lambda i: (0, 0)),
            pl.BlockSpec((1, _H), lambda i: (0, 0)),
            pl.BlockSpec((_H, 1), lambda i: (0, 0)),
            pl.BlockSpec((_H, 1), lambda i: (0, 0)),
            pl.BlockSpec((1, 1), lambda i: (0, 0)),
        ],
        out_specs=[
            pl.BlockSpec((_BLK, 1), lambda i: (i, 0)),
            pl.BlockSpec((_BLK, _H), lambda i: (i, 0)),
        ],
        out_shape=[
            jax.ShapeDtypeStruct((_N, 1), jnp.float32),
            jax.ShapeDtypeStruct((_N, _H), jnp.float32),
        ],
    )(p, x, h, wiht, whht, bih, bhh, wgt, bg, waxt, wagt, ba)


def kernel(x, edge_index, h, W1, b1, W_ih, W_hh, b_ih, b_hh, Wg, bg, Wa, ba):
    npad = _EPAD - _E
    pad_col = _N + jax.lax.iota(jnp.int32, npad) % (_NPAD - _N)
    pad_row = jax.lax.iota(jnp.int32, npad) % _N
    row3 = jnp.concatenate([edge_index[0], pad_row]).reshape(_NW, _NCHUNK, _C)
    col3 = jnp.concatenate([edge_index[1], pad_col]).reshape(_NW, _NCHUNK, _C)
    y = _node_mlp(x, W1.T, b1.reshape(1, _H))
    p = _seg_sum_sc(y, row3, col3)
    a, h_new = _gru_head(
        p, x, h,
        W_ih.T, W_hh.T, b_ih.reshape(1, 3 * _H), b_hh.reshape(1, 3 * _H),
        Wg.T, bg.reshape(1, _H),
        Wa[:, :_H].T, Wa[:, _H:].T, ba.reshape(1, 1))
    return (a, h_new)


# speedup vs baseline: 4.1786x; 1.0042x over previous
import functools

import jax
import jax.numpy as jnp
from jax import lax
from jax.experimental import pallas as pl
from jax.experimental.pallas import tpu as pltpu
from jax.experimental.pallas import tpu_sc as plsc

_N = 10000
_E = 320000
_H = 128

_NC = 2
_NS = 16
_NW = _NC * _NS
_C = 80
_NCHUNK = 125
_EPW = _NCHUNK * _C
_NPAD = 10240
_ZC = 64
_ZPS = (_NPAD // _ZC) // _NS
_WC = 128
_WPS = (_NPAD // _WC) // _NS

_BLK = 1000


def _mlp_body(x_ref, w_ref, b_ref, y_ref):
    y = jnp.dot(x_ref[...], w_ref[...], preferred_element_type=jnp.float32)
    y_ref[...] = jnp.maximum(y + b_ref[...], 0.0)


def _node_mlp(x, w1t, b1):
    return pl.pallas_call(
        _mlp_body,
        grid=(_N // _BLK,),
        in_specs=[
            pl.BlockSpec((_BLK, _H), lambda i: (i, 0)),
            pl.BlockSpec((_H, _H), lambda i: (0, 0)),
            pl.BlockSpec((1, _H), lambda i: (0, 0)),
        ],
        out_specs=pl.BlockSpec((_BLK, _H), lambda i: (i, 0)),
        out_shape=jax.ShapeDtypeStruct((_N, _H), jnp.float32),
    )(x, w1t, b1)


def _seg_sum_body(y_hbm, row_hbm, col_hbm, out_hbm,
                  ridx0, ridx1, ridx2, ridx3, cidx0, cidx1, cidx2, cidx3,
                  rows0, rows1, rows2, rows3, acc,
                  sg0, sg1, sg2, sg3, sr0, sr1, sr2, sr3,
                  sc0, sc1, sc2, sc3, ss0, ss1, ss2, ss3):
    c = lax.axis_index("c")
    s = lax.axis_index("s")
    wid = s * _NC + c
    ridx = [ridx0, ridx1, ridx2, ridx3]
    cidx = [cidx0, cidx1, cidx2, cidx3]
    rows = [rows0, rows1, rows2, rows3]
    sgs = [sg0, sg1, sg2, sg3]
    srs = [sr0, sr1, sr2, sr3]
    scs = [sc0, sc1, sc2, sc3]
    sss = [ss0, ss1, ss2, ss3]

    pltpu.async_copy(row_hbm.at[wid, 0], ridx0, sr0)
    pltpu.async_copy(row_hbm.at[wid, 1], ridx1, sr1)
    pltpu.async_copy(row_hbm.at[wid, 2], ridx2, sr2)
    pltpu.async_copy(row_hbm.at[wid, 3], ridx3, sr3)
    pltpu.async_copy(col_hbm.at[wid, 0], cidx0, sc0)
    pltpu.async_copy(col_hbm.at[wid, 1], cidx1, sc1)

    zero16 = jnp.zeros((16,), jnp.float32)

    def zrows(i, carry):
        rows3[i // (_H // 16), pl.ds((i % (_H // 16)) * 16, 16)] = zero16
        return carry

    lax.fori_loop(0, _C * (_H // 16), zrows, 0)

    def zacc(k, carry):
        j = s * _ZPS + k
        pltpu.sync_copy(rows3.at[pl.ds(0, _ZC)], acc.at[pl.ds(j * _ZC, _ZC)])
        return carry

    lax.fori_loop(0, _ZPS, zacc, 0)

    pltpu.make_async_copy(row_hbm.at[wid, 0], ridx0, sr0).wait()
    pltpu.async_copy(y_hbm.at[ridx0], rows0, sg0)
    pltpu.make_async_copy(row_hbm.at[wid, 0], ridx1, sr1).wait()
    pltpu.async_copy(y_hbm.at[ridx1], rows1, sg1)
    pltpu.make_async_copy(row_hbm.at[wid, 0], ridx2, sr2).wait()
    pltpu.async_copy(y_hbm.at[ridx2], rows2, sg2)
    plsc.subcore_barrier()

    def _chunk(j, m):
        n3 = (m + 3) % 4
        pltpu.make_async_copy(y_hbm.at[ridx[m]], rows[m], sgs[m]).wait()

        @pl.when(j >= 1)
        def _():
            pltpu.make_async_copy(rows[n3], acc.at[cidx[n3]], sss[n3]).wait()

        @pl.when(j + 3 < _NCHUNK)
        def _():
            pltpu.make_async_copy(row_hbm.at[wid, 0], ridx[n3], srs[n3]).wait()
            pltpu.async_copy(y_hbm.at[ridx[n3]], rows[n3], sgs[n3])

        pltpu.make_async_copy(col_hbm.at[wid, 0], cidx[m], scs[m]).wait()
        pltpu.async_copy(rows[m], acc.at[cidx[m]], sss[m], add=True)

        @pl.when(j + 4 < _NCHUNK)
        def _():
            pltpu.async_copy(row_hbm.at[wid, j + 4], ridx[m], srs[m])

        @pl.when(j + 2 < _NCHUNK)
        def _():
            pltpu.async_copy(col_hbm.at[wid, j + 2], cidx[(m + 2) % 4], scs[(m + 2) % 4])

    def quad(q, carry):
        j0 = q * 4
        for m in range(4):
            _chunk(j0 + m, m)
        return carry

    lax.fori_loop(0, _NCHUNK // 4, quad, 0)
    _chunk(jnp.int32(_NCHUNK - 1), 0)
    pltpu.make_async_copy(rows[0], acc.at[cidx[0]], sss[0]).wait()
    plsc.subcore_barrier()

    def wout(k, carry):
        j = s * _WPS + k
        pltpu.sync_copy(acc.at[pl.ds(j * _WC, _WC)], out_hbm.at[c, pl.ds(j * _WC, _WC)])
        return carry

    lax.fori_loop(0, _WPS, wout, 0)


def _seg_sum_sc(y, row3, col3):
    mesh = plsc.VectorSubcoreMesh(
        core_axis_name="c", subcore_axis_name="s",
        num_cores=_NC, num_subcores=_NS)
    f = functools.partial(
        pl.kernel,
        mesh=mesh,
        out_type=jax.ShapeDtypeStruct((_NC, _NPAD, _H), jnp.float32),
        scratch_types=(
            [pltpu.VMEM((_C,), jnp.int32)] * 8
            + [pltpu.VMEM((_C, _H), jnp.float32)] * 4
            + [pltpu.VMEM_SHARED((_NPAD, _H), jnp.float32)]
            + [pltpu.SemaphoreType.DMA] * 16
        ),
    )(_seg_sum_body)
    return f(y, row3, col3)


def _gru_head_body(p_ref, x_ref, h_ref, wih_ref, whh_ref, bih_ref, bhh_ref,
                   wg_ref, bg_ref, wax_ref, wag_ref, ba_ref, a_ref, hn_ref):
    xt = p_ref[0] + p_ref[1]
    h0 = h_ref[...]
    gi = jnp.dot(xt, wih_ref[...], preferred_element_type=jnp.float32) + bih_ref[...]
    gh = jnp.dot(h0, whh_ref[...], preferred_element_type=jnp.float32) + bhh_ref[...]
    r = jax.nn.sigmoid(gi[:, :_H] + gh[:, :_H])
    z = jax.nn.sigmoid(gi[:, _H:2 * _H] + gh[:, _H:2 * _H])
    n = jnp.tanh(gi[:, 2 * _H:] + r * gh[:, 2 * _H:])
    hn = (1.0 - z) * n + z * h0
    hn_ref[...] = hn
    g = jnp.maximum(
        jnp.dot(hn, wg_ref[...], preferred_element_type=jnp.float32) + bg_ref[...], 0.0)
    sacc = (jnp.dot(x_ref[...], wax_ref[...], preferred_element_type=jnp.float32)
            + jnp.dot(g, wag_ref[...], preferred_element_type=jnp.float32)
            + ba_ref[...])
    a_ref[...] = jax.nn.softplus(sacc)


def _gru_head(p, x, h, wiht, whht, bih, bhh, wgt, bg, waxt, wagt, ba):
    return pl.pallas_call(
        _gru_head_body,
        grid=(_N // _BLK,),
        in_specs=[
            pl.BlockSpec((_NC, _BLK, _H), lambda i: (0, i, 0)),
            pl.BlockSpec((_BLK, _H), lambda i: (i, 0)),
            pl.BlockSpec((_BLK, _H), lambda i: (i, 0)),
            pl.BlockSpec((_H, 3 * _H), lambda i: (0, 0)),
            pl.BlockSpec((_H, 3 * _H), lambda i: (0, 0)),
            pl.BlockSpec((1, 3 * _H), lambda i: (0, 0)),
            pl.BlockSpec((1, 3 * _H), lambda i: (0, 0)),
            pl.BlockSpec((_H, _H), lambda i: (0, 0)),
            pl.BlockSpec((1, _H), lambda i: (0, 0)),
            pl.BlockSpec((_H, 1), lambda i: (0, 0)),
            pl.BlockSpec((_H, 1), lambda i: (0, 0)),
            pl.BlockSpec((1, 1), lambda i: (0, 0)),
        ],
        out_specs=[
            pl.BlockSpec((_BLK, 1), lambda i: (i, 0)),
            pl.BlockSpec((_BLK, _H), lambda i: (i, 0)),
        ],
        out_shape=[
            jax.ShapeDtypeStruct((_N, 1), jnp.float32),
            jax.ShapeDtypeStruct((_N, _H), jnp.float32),
        ],
    )(p, x, h, wiht, whht, bih, bhh, wgt, bg, waxt, wagt, ba)


def kernel(x, edge_index, h, W1, b1, W_ih, W_hh, b_ih, b_hh, Wg, bg, Wa, ba):
    row3 = edge_index[0].reshape(_NW, _NCHUNK, _C)
    col3 = edge_index[1].reshape(_NW, _NCHUNK, _C)
    y = _node_mlp(x, W1.T, b1.reshape(1, _H))
    p = _seg_sum_sc(y, row3, col3)
    a, h_new = _gru_head(
        p, x, h,
        W_ih.T, W_hh.T, b_ih.reshape(1, 3 * _H), b_hh.reshape(1, 3 * _H),
        Wg.T, bg.reshape(1, _H),
        Wa[:, :_H].T, Wa[:, _H:].T, ba.reshape(1, 1))
    return (a, h_new)
